# baseline (speedup 1.0000x reference)
import jax
import jax.numpy as jnp
from jax.experimental import pallas as pl
from jax.experimental.pallas import tpu as pltpu

_N = 2048
_D = 128
_NCHUNK = 8
_CH = _N // _NCHUNK


def _fused_kernel(a_hbm, x_ref, w0_ref, w1_ref, gf_ref,
                  fc1w_ref, fc1b_ref, fc2w_ref, fc2b_ref,
                  fc3w_ref, fc3b_ref, fc4w_ref, fc4b_ref, out_ref,
                  a_vmem, y1_vmem, h2_vmem, sems):
    for c in range(_NCHUNK):
        pltpu.make_async_copy(
            a_hbm.at[pl.ds(c * _CH, _CH), :],
            a_vmem.at[pl.ds(c * _CH, _CH), :],
            sems.at[c],
        ).start()
    y0 = jnp.dot(x_ref[...], w0_ref[...], preferred_element_type=jnp.float32)
    w1 = w1_ref[...]
    last = _NCHUNK - 1
    s = jnp.zeros((1, _D), jnp.float32)
    sq = jnp.zeros((1, _D), jnp.float32)
    mx = jnp.full((1, _D), -jnp.inf, jnp.float32)
    mn = jnp.full((1, _D), jnp.inf, jnp.float32)
    for k in range(_NCHUNK):
        rows = pl.ds(k * _CH, _CH)
        pltpu.make_async_copy(
            a_hbm.at[rows, :], a_vmem.at[rows, :], sems.at[k],
        ).wait()
        h1 = jax.nn.relu(jnp.dot(a_vmem[rows, :], y0,
                                 preferred_element_type=jnp.float32))
        y1_k = jnp.dot(h1, w1, preferred_element_type=jnp.float32)
        y1_vmem[rows, :] = y1_k
        if k > 0:
            h2_vmem[rows, :] = jnp.dot(
                a_vmem[rows, pl.ds(0, k * _CH)],
                y1_vmem[pl.ds(0, k * _CH), :],
                preferred_element_type=jnp.float32)
        if k < last:
            span = pl.ds(0, (k + 1) * _CH)
            cols = pl.ds(k * _CH, _CH)
            part = jnp.dot(a_vmem[span, cols], y1_k,
                           preferred_element_type=jnp.float32)
            if k == 0:
                h2_vmem[span, :] = part
            else:
                h2_vmem[span, :] = h2_vmem[span, :] + part
        else:
            cols = pl.ds(k * _CH, _CH)
            for r in range(_NCHUNK):
                rr = pl.ds(r * _CH, _CH)
                h = jax.nn.relu(
                    h2_vmem[rr, :]
                    + jnp.dot(a_vmem[rr, cols], y1_k,
                              preferred_element_type=jnp.float32))
                s = s + jnp.sum(h, axis=0, keepdims=True)
                sq = sq + jnp.sum(jnp.square(h), axis=0, keepdims=True)
                mx = jnp.maximum(mx, jnp.max(h, axis=0, keepdims=True))
                mn = jnp.minimum(mn, jnp.min(h, axis=0, keepdims=True))
    avg = s / _N
    var = sq / _N - jnp.square(avg)
    var = jnp.where(var <= 0.0, jnp.float32(1e-06), var)
    std = jnp.sqrt(var)
    z = jnp.concatenate((avg, std, mx, mn), axis=1)
    z = (jnp.dot(z, fc1w_ref[:4 * _D, :], preferred_element_type=jnp.float32)
         + jnp.dot(gf_ref[...], fc1w_ref[4 * _D:, :],
                   preferred_element_type=jnp.float32)
         + fc1b_ref[...].reshape(1, -1))
    z = jax.nn.relu(z)
    z = jax.nn.relu(jnp.dot(z, fc2w_ref[...],
                            preferred_element_type=jnp.float32)
                    + fc2b_ref[...].reshape(1, -1))
    z = jax.nn.relu(jnp.dot(z, fc3w_ref[...],
                            preferred_element_type=jnp.float32)
                    + fc3b_ref[...].reshape(1, -1))
    z = (jnp.dot(z, fc4w_ref[...], preferred_element_type=jnp.float32)
         + fc4b_ref[...].reshape(1, -1))
    col = jax.lax.broadcasted_iota(jnp.int32, z.shape, 1)
    half = z.shape[1] // 2
    out_ref[...] = jnp.where(col >= half, jnp.square(z), z)


def kernel(x_0, x_1, x_2, x_3, x_4, n0_to_0, n1_to_1, n2_to_2, n3_to_3,
           n4_to_4, n0_to_1, n0_to_2, n0_to_3, n0_to_4, n1_to_2, n1_to_3,
           n1_to_4, n2_to_3, n2_to_4, n3_to_4, global_feature,
           W0_0, W0_1, W0_2, W0_3, W0_4, W1_0, W1_1, W1_2, W1_3, W1_4,
           fc1_w, fc1_b, fc2_w, fc2_b, fc3_w, fc3_b, fc4_w, fc4_b):
    out = pl.pallas_call(
        _fused_kernel,
        out_shape=jax.ShapeDtypeStruct((1, 2), jnp.float32),
        in_specs=[pl.BlockSpec(memory_space=pltpu.MemorySpace.HBM)] +
                 [pl.BlockSpec(memory_space=pltpu.MemorySpace.VMEM)] * 12,
        scratch_shapes=[
            pltpu.MemorySpace.VMEM((_N, _N), jnp.float32),
            pltpu.MemorySpace.VMEM((_N, _D), jnp.float32),
            pltpu.MemorySpace.VMEM((_N, _D), jnp.float32),
            pltpu.SemaphoreType.DMA((_NCHUNK,)),
        ],
    )(n0_to_0, x_0, W0_0, W1_0, global_feature,
      fc1_w, fc1_b, fc2_w, fc2_b, fc3_w, fc3_b, fc4_w, fc4_b)
    return out


# speedup vs baseline: 1.1052x; 1.1052x over previous
import jax
import jax.numpy as jnp
from jax.experimental import pallas as pl
from jax.experimental.pallas import tpu as pltpu

_N = 2048
_D = 128
_NCHUNK = 8
_CH = _N // _NCHUNK


def _fused_kernel(a_hbm, x_ref, w0_ref, w1_ref, gf_ref,
                  fc1w_ref, fc1b_ref, fc2w_ref, fc2b_ref,
                  fc3w_ref, fc3b_ref, fc4w_ref, fc4b_ref, out_ref,
                  a_vmem, y1_vmem, sems):
    for c in range(_NCHUNK):
        pltpu.make_async_copy(
            a_hbm.at[pl.ds(c * _CH, _CH), :],
            a_vmem.at[pl.ds(c * _CH, _CH), :],
            sems.at[c],
        ).start()
    y0 = jnp.dot(x_ref[...], w0_ref[...], preferred_element_type=jnp.float32)
    w1 = w1_ref[...]
    for c in range(_NCHUNK):
        rows = pl.ds(c * _CH, _CH)
        pltpu.make_async_copy(
            a_hbm.at[rows, :], a_vmem.at[rows, :], sems.at[c],
        ).wait()
        h1 = jax.nn.relu(jnp.dot(a_vmem[rows, :], y0,
                                 preferred_element_type=jnp.float32))
        y1_vmem[rows, :] = jnp.dot(h1, w1,
                                   preferred_element_type=jnp.float32)
    y1 = y1_vmem[...]
    s = jnp.zeros((1, _D), jnp.float32)
    sq = jnp.zeros((1, _D), jnp.float32)
    mx = jnp.full((1, _D), -jnp.inf, jnp.float32)
    mn = jnp.full((1, _D), jnp.inf, jnp.float32)
    for r in range(2):
        rows = pl.ds(r * (_N // 2), _N // 2)
        h = jax.nn.relu(jnp.dot(a_vmem[rows, :], y1,
                                preferred_element_type=jnp.float32))
        s = s + jnp.sum(h, axis=0, keepdims=True)
        sq = sq + jnp.sum(jnp.square(h), axis=0, keepdims=True)
        mx = jnp.maximum(mx, jnp.max(h, axis=0, keepdims=True))
        mn = jnp.minimum(mn, jnp.min(h, axis=0, keepdims=True))
    avg = s / _N
    var = sq / _N - jnp.square(avg)
    var = jnp.where(var <= 0.0, jnp.float32(1e-06), var)
    std = jnp.sqrt(var)
    z = jnp.concatenate((avg, std, mx, mn), axis=1)
    z = (jnp.dot(z, fc1w_ref[:4 * _D, :], preferred_element_type=jnp.float32)
         + jnp.dot(gf_ref[...], fc1w_ref[4 * _D:, :],
                   preferred_element_type=jnp.float32)
         + fc1b_ref[...].reshape(1, -1))
    z = jax.nn.relu(z)
    z = jax.nn.relu(jnp.dot(z, fc2w_ref[...],
                            preferred_element_type=jnp.float32)
                    + fc2b_ref[...].reshape(1, -1))
    z = jax.nn.relu(jnp.dot(z, fc3w_ref[...],
                            preferred_element_type=jnp.float32)
                    + fc3b_ref[...].reshape(1, -1))
    z = (jnp.dot(z, fc4w_ref[...], preferred_element_type=jnp.float32)
         + fc4b_ref[...].reshape(1, -1))
    col = jax.lax.broadcasted_iota(jnp.int32, z.shape, 1)
    half = z.shape[1] // 2
    out_ref[...] = jnp.where(col >= half, jnp.square(z), z)


def kernel(x_0, x_1, x_2, x_3, x_4, n0_to_0, n1_to_1, n2_to_2, n3_to_3,
           n4_to_4, n0_to_1, n0_to_2, n0_to_3, n0_to_4, n1_to_2, n1_to_3,
           n1_to_4, n2_to_3, n2_to_4, n3_to_4, global_feature,
           W0_0, W0_1, W0_2, W0_3, W0_4, W1_0, W1_1, W1_2, W1_3, W1_4,
           fc1_w, fc1_b, fc2_w, fc2_b, fc3_w, fc3_b, fc4_w, fc4_b):
    out = pl.pallas_call(
        _fused_kernel,
        out_shape=jax.ShapeDtypeStruct((1, 2), jnp.float32),
        in_specs=[pl.BlockSpec(memory_space=pltpu.MemorySpace.HBM)] +
                 [pl.BlockSpec(memory_space=pltpu.MemorySpace.VMEM)] * 12,
        scratch_shapes=[
            pltpu.MemorySpace.VMEM((_N, _N), jnp.float32),
            pltpu.MemorySpace.VMEM((_N, _D), jnp.float32),
            pltpu.SemaphoreType.DMA((_NCHUNK,)),
        ],
    )(n0_to_0, x_0, W0_0, W1_0, global_feature,
      fc1_w, fc1_b, fc2_w, fc2_b, fc3_w, fc3_b, fc4_w, fc4_b)
    return out


# speedup vs baseline: 1.1672x; 1.0561x over previous
import jax
import jax.numpy as jnp
from jax.experimental import pallas as pl
from jax.experimental.pallas import tpu as pltpu

_N = 2048
_D = 128
_NCHUNK = 8
_CH = _N // _NCHUNK


def _fused_kernel(a_hbm, x_ref, w0_ref, w1_ref, gf_ref,
                  fc1w_ref, fc1b_ref, fc2w_ref, fc2b_ref,
                  fc3w_ref, fc3b_ref, fc4w_ref, fc4b_ref, out_ref,
                  a_vmem, y1_vmem, sems):
    for c in range(_NCHUNK):
        pltpu.make_async_copy(
            a_hbm.at[pl.ds(c * _CH, _CH), :],
            a_vmem.at[pl.ds(c * _CH, _CH), :],
            sems.at[c],
        ).start()
    y0 = jnp.dot(x_ref[...], w0_ref[...], preferred_element_type=jnp.float32)
    w1 = w1_ref[...]
    for c in range(_NCHUNK):
        rows = pl.ds(c * _CH, _CH)
        pltpu.make_async_copy(
            a_hbm.at[rows, :], a_vmem.at[rows, :], sems.at[c],
        ).wait()
        y1_vmem[rows, :] = jax.nn.relu(
            jnp.dot(a_vmem[rows, :], y0, preferred_element_type=jnp.float32))
    y1 = jnp.dot(y1_vmem[...], w1, preferred_element_type=jnp.float32)
    h = jax.nn.relu(jnp.dot(a_vmem[...], y1,
                            preferred_element_type=jnp.float32))
    avg = jnp.sum(h, axis=0, keepdims=True) / _N
    var = jnp.sum(jnp.square(h), axis=0, keepdims=True) / _N - jnp.square(avg)
    mx = jnp.max(h, axis=0, keepdims=True)
    mn = jnp.min(h, axis=0, keepdims=True)
    var = jnp.where(var <= 0.0, jnp.float32(1e-06), var)
    std = jnp.sqrt(var)
    z = jnp.concatenate((avg, std, mx, mn), axis=1)
    z = (jnp.dot(z, fc1w_ref[:4 * _D, :], preferred_element_type=jnp.float32)
         + jnp.dot(gf_ref[...], fc1w_ref[4 * _D:, :],
                   preferred_element_type=jnp.float32)
         + fc1b_ref[...].reshape(1, -1))
    z = jax.nn.relu(z)
    z = jax.nn.relu(jnp.dot(z, fc2w_ref[...],
                            preferred_element_type=jnp.float32)
                    + fc2b_ref[...].reshape(1, -1))
    z = jax.nn.relu(jnp.dot(z, fc3w_ref[...],
                            preferred_element_type=jnp.float32)
                    + fc3b_ref[...].reshape(1, -1))
    z = (jnp.dot(z, fc4w_ref[...], preferred_element_type=jnp.float32)
         + fc4b_ref[...].reshape(1, -1))
    col = jax.lax.broadcasted_iota(jnp.int32, z.shape, 1)
    half = z.shape[1] // 2
    out_ref[...] = jnp.where(col >= half, jnp.square(z), z)


def kernel(x_0, x_1, x_2, x_3, x_4, n0_to_0, n1_to_1, n2_to_2, n3_to_3,
           n4_to_4, n0_to_1, n0_to_2, n0_to_3, n0_to_4, n1_to_2, n1_to_3,
           n1_to_4, n2_to_3, n2_to_4, n3_to_4, global_feature,
           W0_0, W0_1, W0_2, W0_3, W0_4, W1_0, W1_1, W1_2, W1_3, W1_4,
           fc1_w, fc1_b, fc2_w, fc2_b, fc3_w, fc3_b, fc4_w, fc4_b):
    out = pl.pallas_call(
        _fused_kernel,
        out_shape=jax.ShapeDtypeStruct((1, 2), jnp.float32),
        in_specs=[pl.BlockSpec(memory_space=pltpu.MemorySpace.HBM)] +
                 [pl.BlockSpec(memory_space=pltpu.MemorySpace.VMEM)] * 12,
        scratch_shapes=[
            pltpu.MemorySpace.VMEM((_N, _N), jnp.float32),
            pltpu.MemorySpace.VMEM((_N, _D), jnp.float32),
            pltpu.SemaphoreType.DMA((_NCHUNK,)),
        ],
    )(n0_to_0, x_0, W0_0, W1_0, global_feature,
      fc1_w, fc1_b, fc2_w, fc2_b, fc3_w, fc3_b, fc4_w, fc4_b)
    return out


# speedup vs baseline: 1.1715x; 1.0037x over previous
import jax
import jax.numpy as jnp
from jax.experimental import pallas as pl
from jax.experimental.pallas import tpu as pltpu

_N = 2048
_D = 128
_NCHUNK = 16
_CH = _N // _NCHUNK


def _fused_kernel(a_hbm, x_ref, w0_ref, w1_ref, gf_ref,
                  fc1w_ref, fc1b_ref, fc2w_ref, fc2b_ref,
                  fc3w_ref, fc3b_ref, fc4w_ref, fc4b_ref, out_ref,
                  a_vmem, y1_vmem, sems):
    for c in range(_NCHUNK):
        pltpu.make_async_copy(
            a_hbm.at[pl.ds(c * _CH, _CH), :],
            a_vmem.at[pl.ds(c * _CH, _CH), :],
            sems.at[c],
        ).start()
    y0 = jnp.dot(x_ref[...], w0_ref[...], preferred_element_type=jnp.float32)
    w1 = w1_ref[...]
    for c in range(_NCHUNK):
        rows = pl.ds(c * _CH, _CH)
        pltpu.make_async_copy(
            a_hbm.at[rows, :], a_vmem.at[rows, :], sems.at[c],
        ).wait()
        y1_vmem[rows, :] = jax.nn.relu(
            jnp.dot(a_vmem[rows, :], y0, preferred_element_type=jnp.float32))
    y1 = jnp.dot(y1_vmem[...], w1, preferred_element_type=jnp.float32)
    h = jax.nn.relu(jnp.dot(a_vmem[...], y1,
                            preferred_element_type=jnp.float32))
    avg = jnp.sum(h, axis=0, keepdims=True) / _N
    var = jnp.sum(jnp.square(h), axis=0, keepdims=True) / _N - jnp.square(avg)
    mx = jnp.max(h, axis=0, keepdims=True)
    mn = jnp.min(h, axis=0, keepdims=True)
    var = jnp.where(var <= 0.0, jnp.float32(1e-06), var)
    std = jnp.sqrt(var)
    z = jnp.concatenate((avg, std, mx, mn), axis=1)
    z = (jnp.dot(z, fc1w_ref[:4 * _D, :], preferred_element_type=jnp.float32)
         + jnp.dot(gf_ref[...], fc1w_ref[4 * _D:, :],
                   preferred_element_type=jnp.float32)
         + fc1b_ref[...].reshape(1, -1))
    z = jax.nn.relu(z)
    z = jax.nn.relu(jnp.dot(z, fc2w_ref[...],
                            preferred_element_type=jnp.float32)
                    + fc2b_ref[...].reshape(1, -1))
    z = jax.nn.relu(jnp.dot(z, fc3w_ref[...],
                            preferred_element_type=jnp.float32)
                    + fc3b_ref[...].reshape(1, -1))
    z = (jnp.dot(z, fc4w_ref[...], preferred_element_type=jnp.float32)
         + fc4b_ref[...].reshape(1, -1))
    col = jax.lax.broadcasted_iota(jnp.int32, z.shape, 1)
    half = z.shape[1] // 2
    out_ref[...] = jnp.where(col >= half, jnp.square(z), z)


def kernel(x_0, x_1, x_2, x_3, x_4, n0_to_0, n1_to_1, n2_to_2, n3_to_3,
           n4_to_4, n0_to_1, n0_to_2, n0_to_3, n0_to_4, n1_to_2, n1_to_3,
           n1_to_4, n2_to_3, n2_to_4, n3_to_4, global_feature,
           W0_0, W0_1, W0_2, W0_3, W0_4, W1_0, W1_1, W1_2, W1_3, W1_4,
           fc1_w, fc1_b, fc2_w, fc2_b, fc3_w, fc3_b, fc4_w, fc4_b):
    out = pl.pallas_call(
        _fused_kernel,
        out_shape=jax.ShapeDtypeStruct((1, 2), jnp.float32),
        in_specs=[pl.BlockSpec(memory_space=pltpu.MemorySpace.HBM)] +
                 [pl.BlockSpec(memory_space=pltpu.MemorySpace.VMEM)] * 12,
        scratch_shapes=[
            pltpu.MemorySpace.VMEM((_N, _N), jnp.float32),
            pltpu.MemorySpace.VMEM((_N, _D), jnp.float32),
            pltpu.SemaphoreType.DMA((_NCHUNK,)),
        ],
    )(n0_to_0, x_0, W0_0, W1_0, global_feature,
      fc1_w, fc1_b, fc2_w, fc2_b, fc3_w, fc3_b, fc4_w, fc4_b)
    return out
